# baseline (speedup 1.0000x reference)
import functools

import jax
import jax.numpy as jnp
from jax import lax
from jax.experimental import pallas as pl
from jax.experimental.pallas import tpu as pltpu
from jax.experimental.pallas import tpu_sc as plsc

N = 10000
E = 320000
D_IN = 128
D_HID = 128
D_OUT = 64

NC = 2
NS = 16
NW = NC * NS
EPT = E // NW
CH = 40
NCH = EPT // CH
NBUF = 6
NIB = 12
RPT = 632
RPT_LAST = N - (NS - 1) * RPT

_mesh = plsc.VectorSubcoreMesh(core_axis_name="c", subcore_axis_name="s")


def _zero_acc(z_hbm, acc, s):
    @pl.when(s < NS - 1)
    def _():
        pltpu.sync_copy(z_hbm, acc.at[pl.ds(s * RPT, RPT)])

    @pl.when(s == NS - 1)
    def _():
        pltpu.sync_copy(z_hbm.at[pl.ds(0, RPT_LAST)],
                        acc.at[pl.ds(s * RPT, RPT_LAST)])


def _acc_to_out(acc, out_hbm, c, s):
    @pl.when(s < NS - 1)
    def _():
        pltpu.sync_copy(acc.at[pl.ds(s * RPT, RPT)],
                        out_hbm.at[c].at[pl.ds(s * RPT, RPT)])

    @pl.when(s == NS - 1)
    def _():
        pltpu.sync_copy(acc.at[pl.ds(s * RPT, RPT_LAST)],
                        out_hbm.at[c].at[pl.ds(s * RPT, RPT_LAST)])


def _make_edge_kernel(d):
    scratch = [pltpu.VMEM_SHARED((N, d), jnp.float32)]
    scratch += [pltpu.VMEM((CH,), jnp.int32)] * NIB
    scratch += [pltpu.VMEM((CH,), jnp.int32)] * NIB
    scratch += [pltpu.VMEM((CH, d), jnp.float32)] * NBUF
    scratch += [pltpu.SemaphoreType.DMA] * (NIB + NBUF + NBUF)

    @functools.partial(
        pl.kernel,
        out_type=jax.ShapeDtypeStruct((NC, N, d), jnp.float32),
        mesh=_mesh,
        scratch_types=scratch,
        name=f"sc_edge_agg_{d}",
    )
    def edge_kernel(g_hbm, src_hbm, dst_hbm, z_hbm, out_hbm, acc, *bufs):
        sv = bufs[0:NIB]
        dv = bufs[NIB:2 * NIB]
        rv = bufs[2 * NIB:2 * NIB + NBUF]
        si = bufs[2 * NIB + NBUF:2 * NIB + NBUF + NIB]
        sg = bufs[2 * NIB + NBUF + NIB:2 * NIB + NBUF + NIB + NBUF]
        ss = bufs[2 * NIB + NBUF + NIB + NBUF:]
        c = lax.axis_index("c")
        s = lax.axis_index("s")
        wid = s * NC + c
        base = wid * EPT

        def idx_fetch(j, ib):
            pltpu.async_copy(src_hbm.at[pl.ds(base + j * CH, CH)],
                             sv[ib], si[ib])
            pltpu.async_copy(dst_hbm.at[pl.ds(base + j * CH, CH)],
                             dv[ib], si[ib])

        def idx_wait(j, ib):
            pltpu.make_async_copy(src_hbm.at[pl.ds(base + j * CH, CH)],
                                  sv[ib], si[ib]).wait()
            pltpu.make_async_copy(dst_hbm.at[pl.ds(base + j * CH, CH)],
                                  dv[ib], si[ib]).wait()

        _zero_acc(z_hbm, acc, s)
        for k in range(NBUF):
            idx_fetch(k, k)
        plsc.subcore_barrier()

        def step(j, k):
            b = k % NBUF
            ib = k % NIB
            pb = (k - 1) % NBUF
            pib = (k - 1) % NIB
            fib = (k + NBUF) % NIB

            @pl.when(j >= NBUF)
            def _():
                pltpu.make_async_copy(rv[b], acc.at[dv[(k - NBUF) % NIB]],
                                      ss[b]).wait()

            @pl.when(j + NBUF < NCH)
            def _():
                idx_fetch(j + NBUF, fib)

            idx_wait(j, ib)
            pltpu.async_copy(g_hbm.at[sv[ib]], rv[b], sg[b])

            @pl.when(j >= 1)
            def _():
                pltpu.make_async_copy(g_hbm.at[sv[pib]], rv[pb],
                                      sg[pb]).wait()
                pltpu.async_copy(rv[pb], acc.at[dv[pib]], ss[pb], add=True)

        def outer(j2, carry):
            for k in range(NIB):
                j = j2 * NIB + k

                @pl.when(j < NCH)
                def _():
                    step(j, k)
            return carry

        lax.fori_loop(0, (NCH + NIB - 1) // NIB, outer, 0)

        last = NCH - 1
        lb, lib = last % NBUF, last % NIB
        pltpu.make_async_copy(g_hbm.at[sv[lib]], rv[lb], sg[lb]).wait()
        pltpu.async_copy(rv[lb], acc.at[dv[lib]], ss[lb], add=True)
        for jj in range(NCH - NBUF, NCH):
            pltpu.make_async_copy(rv[jj % NBUF], acc.at[dv[jj % NIB]],
                                  ss[jj % NBUF]).wait()

        plsc.subcore_barrier()
        _acc_to_out(acc, out_hbm, c, s)

    return edge_kernel


_edge128 = _make_edge_kernel(D_HID)


_DEG_DEPTH = 4
_DEG_NIB = 8


@functools.partial(
    pl.kernel,
    out_type=jax.ShapeDtypeStruct((NC, N, D_HID), jnp.float32),
    mesh=_mesh,
    scratch_types=(
        [pltpu.VMEM_SHARED((N, D_HID), jnp.float32)]
        + [pltpu.VMEM((CH,), jnp.int32)] * _DEG_NIB
        + [pltpu.VMEM((CH, D_HID), jnp.float32)]
        + [pltpu.SemaphoreType.DMA] * (2 * _DEG_NIB)
    ),
    name="sc_degree",
)
def _sc_degree(dst_hbm, z_hbm, o_hbm, out_hbm, acc, *bufs):
    dv = bufs[0:_DEG_NIB]
    ones_v = bufs[_DEG_NIB]
    si = bufs[_DEG_NIB + 1:2 * _DEG_NIB + 1]
    ss = bufs[2 * _DEG_NIB + 1:]
    c = lax.axis_index("c")
    s = lax.axis_index("s")
    wid = s * NC + c
    base = wid * EPT

    def idx_fetch(j, ib):
        pltpu.async_copy(dst_hbm.at[pl.ds(base + j * CH, CH)], dv[ib], si[ib])

    pltpu.sync_copy(o_hbm, ones_v)
    _zero_acc(z_hbm, acc, s)
    for k in range(_DEG_DEPTH):
        idx_fetch(k, k)
    plsc.subcore_barrier()

    def step(j, k):
        ib = k % _DEG_NIB

        @pl.when(j >= _DEG_DEPTH)
        def _():
            pltpu.make_async_copy(
                ones_v, acc.at[dv[(k - _DEG_DEPTH) % _DEG_NIB]],
                ss[(k - _DEG_DEPTH) % _DEG_NIB]).wait()

        @pl.when(j + _DEG_DEPTH < NCH)
        def _():
            idx_fetch(j + _DEG_DEPTH, (k + _DEG_DEPTH) % _DEG_NIB)

        pltpu.make_async_copy(dst_hbm.at[pl.ds(base + j * CH, CH)],
                              dv[ib], si[ib]).wait()
        pltpu.async_copy(ones_v, acc.at[dv[ib]], ss[ib], add=True)

    def outer(j2, carry):
        for k in range(_DEG_NIB):
            j = j2 * _DEG_NIB + k

            @pl.when(j < NCH)
            def _():
                step(j, k)
        return carry

    lax.fori_loop(0, (NCH + _DEG_NIB - 1) // _DEG_NIB, outer, 0)
    for jj in range(NCH - _DEG_DEPTH, NCH):
        pltpu.make_async_copy(ones_v, acc.at[dv[jj % _DEG_NIB]],
                              ss[jj % _DEG_NIB]).wait()

    plsc.subcore_barrier()
    _acc_to_out(acc, out_hbm, c, s)


_BLK = 1000


def _stage0_body(degp_ref, x_ref, w_ref, dis_ref, dis2_ref, p_ref, g_ref):
    deg = degp_ref[0, :, :1] + degp_ref[1, :, :1] + 1.0
    dis = lax.rsqrt(deg)
    dis_ref[...] = dis
    dis2_ref[...] = dis * dis
    p = jnp.dot(x_ref[...], w_ref[...], preferred_element_type=jnp.float32)
    p_ref[...] = p
    g_ref[...] = p * dis


def _tc_stage0(deg_parts, x, w0):
    return pl.pallas_call(
        _stage0_body,
        grid=(N // _BLK,),
        in_specs=[
            pl.BlockSpec((NC, _BLK, D_HID), lambda i: (0, i, 0)),
            pl.BlockSpec((_BLK, D_IN), lambda i: (i, 0)),
            pl.BlockSpec((D_IN, D_HID), lambda i: (0, 0)),
        ],
        out_specs=[
            pl.BlockSpec((_BLK, 1), lambda i: (i, 0)),
            pl.BlockSpec((_BLK, 1), lambda i: (i, 0)),
            pl.BlockSpec((_BLK, D_HID), lambda i: (i, 0)),
            pl.BlockSpec((_BLK, D_HID), lambda i: (i, 0)),
        ],
        out_shape=[
            jax.ShapeDtypeStruct((N, 1), jnp.float32),
            jax.ShapeDtypeStruct((N, 1), jnp.float32),
            jax.ShapeDtypeStruct((N, D_HID), jnp.float32),
            jax.ShapeDtypeStruct((N, D_HID), jnp.float32),
        ],
        name="tc_stage0",
    )(deg_parts, x, w0)


def _combine_body(sp_ref, p_ref, dis_ref, dis2_ref, b_ref, w_ref,
                  pn_ref, gn_ref):
    S = sp_ref[0] + sp_ref[1]
    dis = dis_ref[...]
    h = S * dis + p_ref[...] * dis2_ref[...] + b_ref[...]
    h = jnp.maximum(h, 0.0)
    pn = jnp.dot(h, w_ref[...], preferred_element_type=jnp.float32)
    pn_ref[...] = pn
    gn_ref[...] = pn * dis


def _tc_combine(sp, p, dis, dis2, b, w):
    d_in = p.shape[1]
    d_out = w.shape[1]
    return pl.pallas_call(
        _combine_body,
        grid=(N // _BLK,),
        in_specs=[
            pl.BlockSpec((NC, _BLK, d_in), lambda i: (0, i, 0)),
            pl.BlockSpec((_BLK, d_in), lambda i: (i, 0)),
            pl.BlockSpec((_BLK, 1), lambda i: (i, 0)),
            pl.BlockSpec((_BLK, 1), lambda i: (i, 0)),
            pl.BlockSpec((1, d_in), lambda i: (0, 0)),
            pl.BlockSpec((d_in, d_out), lambda i: (0, 0)),
        ],
        out_specs=[
            pl.BlockSpec((_BLK, d_out), lambda i: (i, 0)),
            pl.BlockSpec((_BLK, d_out), lambda i: (i, 0)),
        ],
        out_shape=[
            jax.ShapeDtypeStruct((N, d_out), jnp.float32),
            jax.ShapeDtypeStruct((N, d_out), jnp.float32),
        ],
        name="tc_combine",
    )(sp, p, dis, dis2, b, w)


def _final_body(sp_ref, p_ref, dis_ref, dis2_ref, b_ref, o_ref):
    S = sp_ref[0] + sp_ref[1]
    full = S * dis_ref[...] + p_ref[...] * dis2_ref[...]
    o_ref[...] = full[:, :D_OUT] + b_ref[...]


def _tc_final(sp, p, dis, dis2, b):
    return pl.pallas_call(
        _final_body,
        grid=(N // _BLK,),
        in_specs=[
            pl.BlockSpec((NC, _BLK, D_HID), lambda i: (0, i, 0)),
            pl.BlockSpec((_BLK, D_HID), lambda i: (i, 0)),
            pl.BlockSpec((_BLK, 1), lambda i: (i, 0)),
            pl.BlockSpec((_BLK, 1), lambda i: (i, 0)),
            pl.BlockSpec((1, D_OUT), lambda i: (0, 0)),
        ],
        out_specs=pl.BlockSpec((_BLK, D_OUT), lambda i: (i, 0)),
        out_shape=jax.ShapeDtypeStruct((N, D_OUT), jnp.float32),
        name="tc_final",
    )(sp, p, dis, dis2, b)


def kernel(x, edge_index, W0, b0, W1, b1, W2, b2):
    src = edge_index[0]
    dst = edge_index[1]
    z128 = jnp.zeros((RPT, D_HID), jnp.float32)
    o128 = jnp.ones((CH, D_HID), jnp.float32)

    deg_parts = _sc_degree(dst, z128, o128)
    dis, dis2, p0, g0 = _tc_stage0(deg_parts, x, W0)
    s0 = _edge128(g0, src, dst, z128)
    p1, g1 = _tc_combine(s0, p0, dis, dis2, b0.reshape(1, -1), W1)
    s1 = _edge128(g1, src, dst, z128)
    w2p = jnp.pad(W2, ((0, 0), (0, D_HID - D_OUT)))
    p2, g2 = _tc_combine(s1, p1, dis, dis2, b1.reshape(1, -1), w2p)
    s2 = _edge128(g2, src, dst, z128)
    return _tc_final(s2, p2, dis, dis2, b2.reshape(1, -1))


# speedup vs baseline: 1.1479x; 1.1479x over previous
import functools

import jax
import jax.numpy as jnp
from jax import lax
from jax.experimental import pallas as pl
from jax.experimental.pallas import tpu as pltpu
from jax.experimental.pallas import tpu_sc as plsc

N = 10000
E = 320000
D_IN = 128
D_HID = 128
D_OUT = 64

NC = 2
NS = 16
NW = NC * NS
EPT = E // NW
CH = 80
NCH = EPT // CH
NBUF = 4
NIB = 8
RPT = 632
RPT_LAST = N - (NS - 1) * RPT

_mesh = plsc.VectorSubcoreMesh(core_axis_name="c", subcore_axis_name="s")


def _zero_acc(z_hbm, acc, s):
    @pl.when(s < NS - 1)
    def _():
        pltpu.sync_copy(z_hbm, acc.at[pl.ds(s * RPT, RPT)])

    @pl.when(s == NS - 1)
    def _():
        pltpu.sync_copy(z_hbm.at[pl.ds(0, RPT_LAST)],
                        acc.at[pl.ds(s * RPT, RPT_LAST)])


def _acc_to_out(acc, out_hbm, c, s):
    @pl.when(s < NS - 1)
    def _():
        pltpu.sync_copy(acc.at[pl.ds(s * RPT, RPT)],
                        out_hbm.at[c].at[pl.ds(s * RPT, RPT)])

    @pl.when(s == NS - 1)
    def _():
        pltpu.sync_copy(acc.at[pl.ds(s * RPT, RPT_LAST)],
                        out_hbm.at[c].at[pl.ds(s * RPT, RPT_LAST)])


def _make_edge_kernel(d):
    scratch = [pltpu.VMEM_SHARED((N, d), jnp.float32)]
    scratch += [pltpu.VMEM((CH,), jnp.int32)] * NIB
    scratch += [pltpu.VMEM((CH,), jnp.int32)] * NIB
    scratch += [pltpu.VMEM((CH, d), jnp.float32)] * NBUF
    scratch += [pltpu.SemaphoreType.DMA] * (NIB + NBUF + NBUF)

    @functools.partial(
        pl.kernel,
        out_type=jax.ShapeDtypeStruct((NC, N, d), jnp.float32),
        mesh=_mesh,
        scratch_types=scratch,
        name=f"sc_edge_agg_{d}",
    )
    def edge_kernel(g_hbm, src_hbm, dst_hbm, z_hbm, out_hbm, acc, *bufs):
        sv = bufs[0:NIB]
        dv = bufs[NIB:2 * NIB]
        rv = bufs[2 * NIB:2 * NIB + NBUF]
        si = bufs[2 * NIB + NBUF:2 * NIB + NBUF + NIB]
        sg = bufs[2 * NIB + NBUF + NIB:2 * NIB + NBUF + NIB + NBUF]
        ss = bufs[2 * NIB + NBUF + NIB + NBUF:]
        c = lax.axis_index("c")
        s = lax.axis_index("s")
        wid = s * NC + c
        base = wid * EPT

        def idx_fetch(j, ib):
            pltpu.async_copy(src_hbm.at[pl.ds(base + j * CH, CH)],
                             sv[ib], si[ib])
            pltpu.async_copy(dst_hbm.at[pl.ds(base + j * CH, CH)],
                             dv[ib], si[ib])

        def idx_wait(j, ib):
            pltpu.make_async_copy(src_hbm.at[pl.ds(base + j * CH, CH)],
                                  sv[ib], si[ib]).wait()
            pltpu.make_async_copy(dst_hbm.at[pl.ds(base + j * CH, CH)],
                                  dv[ib], si[ib]).wait()

        _zero_acc(z_hbm, acc, s)
        for k in range(NBUF):
            idx_fetch(k, k)
        plsc.subcore_barrier()

        def step(j, k):
            b = k % NBUF
            ib = k % NIB
            pb = (k - 1) % NBUF
            pib = (k - 1) % NIB
            fib = (k + NBUF) % NIB

            @pl.when(j >= NBUF)
            def _():
                pltpu.make_async_copy(rv[b], acc.at[dv[(k - NBUF) % NIB]],
                                      ss[b]).wait()

            @pl.when(j + NBUF < NCH)
            def _():
                idx_fetch(j + NBUF, fib)

            idx_wait(j, ib)
            pltpu.async_copy(g_hbm.at[sv[ib]], rv[b], sg[b])

            @pl.when(j >= 1)
            def _():
                pltpu.make_async_copy(g_hbm.at[sv[pib]], rv[pb],
                                      sg[pb]).wait()
                pltpu.async_copy(rv[pb], acc.at[dv[pib]], ss[pb], add=True)

        def outer(j2, carry):
            for k in range(NIB):
                j = j2 * NIB + k

                @pl.when(j < NCH)
                def _():
                    step(j, k)
            return carry

        lax.fori_loop(0, (NCH + NIB - 1) // NIB, outer, 0)

        last = NCH - 1
        lb, lib = last % NBUF, last % NIB
        pltpu.make_async_copy(g_hbm.at[sv[lib]], rv[lb], sg[lb]).wait()
        pltpu.async_copy(rv[lb], acc.at[dv[lib]], ss[lb], add=True)
        for jj in range(NCH - NBUF, NCH):
            pltpu.make_async_copy(rv[jj % NBUF], acc.at[dv[jj % NIB]],
                                  ss[jj % NBUF]).wait()

        plsc.subcore_barrier()
        _acc_to_out(acc, out_hbm, c, s)

    return edge_kernel


_edge128 = _make_edge_kernel(D_HID)


_DEG_DEPTH = 4
_DEG_NIB = 8


@functools.partial(
    pl.kernel,
    out_type=jax.ShapeDtypeStruct((NC, N, D_HID), jnp.float32),
    mesh=_mesh,
    scratch_types=(
        [pltpu.VMEM_SHARED((N, D_HID), jnp.float32)]
        + [pltpu.VMEM((CH,), jnp.int32)] * _DEG_NIB
        + [pltpu.VMEM((CH, D_HID), jnp.float32)]
        + [pltpu.SemaphoreType.DMA] * (2 * _DEG_NIB)
    ),
    name="sc_degree",
)
def _sc_degree(dst_hbm, z_hbm, o_hbm, out_hbm, acc, *bufs):
    dv = bufs[0:_DEG_NIB]
    ones_v = bufs[_DEG_NIB]
    si = bufs[_DEG_NIB + 1:2 * _DEG_NIB + 1]
    ss = bufs[2 * _DEG_NIB + 1:]
    c = lax.axis_index("c")
    s = lax.axis_index("s")
    wid = s * NC + c
    base = wid * EPT

    def idx_fetch(j, ib):
        pltpu.async_copy(dst_hbm.at[pl.ds(base + j * CH, CH)], dv[ib], si[ib])

    pltpu.sync_copy(o_hbm, ones_v)
    _zero_acc(z_hbm, acc, s)
    for k in range(_DEG_DEPTH):
        idx_fetch(k, k)
    plsc.subcore_barrier()

    def step(j, k):
        ib = k % _DEG_NIB

        @pl.when(j >= _DEG_DEPTH)
        def _():
            pltpu.make_async_copy(
                ones_v, acc.at[dv[(k - _DEG_DEPTH) % _DEG_NIB]],
                ss[(k - _DEG_DEPTH) % _DEG_NIB]).wait()

        @pl.when(j + _DEG_DEPTH < NCH)
        def _():
            idx_fetch(j + _DEG_DEPTH, (k + _DEG_DEPTH) % _DEG_NIB)

        pltpu.make_async_copy(dst_hbm.at[pl.ds(base + j * CH, CH)],
                              dv[ib], si[ib]).wait()
        pltpu.async_copy(ones_v, acc.at[dv[ib]], ss[ib], add=True)

    def outer(j2, carry):
        for k in range(_DEG_NIB):
            j = j2 * _DEG_NIB + k

            @pl.when(j < NCH)
            def _():
                step(j, k)
        return carry

    lax.fori_loop(0, (NCH + _DEG_NIB - 1) // _DEG_NIB, outer, 0)
    for jj in range(NCH - _DEG_DEPTH, NCH):
        pltpu.make_async_copy(ones_v, acc.at[dv[jj % _DEG_NIB]],
                              ss[jj % _DEG_NIB]).wait()

    plsc.subcore_barrier()
    _acc_to_out(acc, out_hbm, c, s)


_BLK = 1000


def _stage0_body(degp_ref, x_ref, w_ref, dis_ref, dis2_ref, p_ref, g_ref):
    deg = degp_ref[0, :, :1] + degp_ref[1, :, :1] + 1.0
    dis = lax.rsqrt(deg)
    dis_ref[...] = dis
    dis2_ref[...] = dis * dis
    p = jnp.dot(x_ref[...], w_ref[...], preferred_element_type=jnp.float32)
    p_ref[...] = p
    g_ref[...] = p * dis


def _tc_stage0(deg_parts, x, w0):
    return pl.pallas_call(
        _stage0_body,
        grid=(N // _BLK,),
        in_specs=[
            pl.BlockSpec((NC, _BLK, D_HID), lambda i: (0, i, 0)),
            pl.BlockSpec((_BLK, D_IN), lambda i: (i, 0)),
            pl.BlockSpec((D_IN, D_HID), lambda i: (0, 0)),
        ],
        out_specs=[
            pl.BlockSpec((_BLK, 1), lambda i: (i, 0)),
            pl.BlockSpec((_BLK, 1), lambda i: (i, 0)),
            pl.BlockSpec((_BLK, D_HID), lambda i: (i, 0)),
            pl.BlockSpec((_BLK, D_HID), lambda i: (i, 0)),
        ],
        out_shape=[
            jax.ShapeDtypeStruct((N, 1), jnp.float32),
            jax.ShapeDtypeStruct((N, 1), jnp.float32),
            jax.ShapeDtypeStruct((N, D_HID), jnp.float32),
            jax.ShapeDtypeStruct((N, D_HID), jnp.float32),
        ],
        name="tc_stage0",
    )(deg_parts, x, w0)


def _combine_body(sp_ref, p_ref, dis_ref, dis2_ref, b_ref, w_ref,
                  pn_ref, gn_ref):
    S = sp_ref[0] + sp_ref[1]
    dis = dis_ref[...]
    h = S * dis + p_ref[...] * dis2_ref[...] + b_ref[...]
    h = jnp.maximum(h, 0.0)
    pn = jnp.dot(h, w_ref[...], preferred_element_type=jnp.float32)
    pn_ref[...] = pn
    gn_ref[...] = pn * dis


def _tc_combine(sp, p, dis, dis2, b, w):
    d_in = p.shape[1]
    d_out = w.shape[1]
    return pl.pallas_call(
        _combine_body,
        grid=(N // _BLK,),
        in_specs=[
            pl.BlockSpec((NC, _BLK, d_in), lambda i: (0, i, 0)),
            pl.BlockSpec((_BLK, d_in), lambda i: (i, 0)),
            pl.BlockSpec((_BLK, 1), lambda i: (i, 0)),
            pl.BlockSpec((_BLK, 1), lambda i: (i, 0)),
            pl.BlockSpec((1, d_in), lambda i: (0, 0)),
            pl.BlockSpec((d_in, d_out), lambda i: (0, 0)),
        ],
        out_specs=[
            pl.BlockSpec((_BLK, d_out), lambda i: (i, 0)),
            pl.BlockSpec((_BLK, d_out), lambda i: (i, 0)),
        ],
        out_shape=[
            jax.ShapeDtypeStruct((N, d_out), jnp.float32),
            jax.ShapeDtypeStruct((N, d_out), jnp.float32),
        ],
        name="tc_combine",
    )(sp, p, dis, dis2, b, w)


def _final_body(sp_ref, p_ref, dis_ref, dis2_ref, b_ref, o_ref):
    S = sp_ref[0] + sp_ref[1]
    full = S * dis_ref[...] + p_ref[...] * dis2_ref[...]
    o_ref[...] = full[:, :D_OUT] + b_ref[...]


def _tc_final(sp, p, dis, dis2, b):
    return pl.pallas_call(
        _final_body,
        grid=(N // _BLK,),
        in_specs=[
            pl.BlockSpec((NC, _BLK, D_HID), lambda i: (0, i, 0)),
            pl.BlockSpec((_BLK, D_HID), lambda i: (i, 0)),
            pl.BlockSpec((_BLK, 1), lambda i: (i, 0)),
            pl.BlockSpec((_BLK, 1), lambda i: (i, 0)),
            pl.BlockSpec((1, D_OUT), lambda i: (0, 0)),
        ],
        out_specs=pl.BlockSpec((_BLK, D_OUT), lambda i: (i, 0)),
        out_shape=jax.ShapeDtypeStruct((N, D_OUT), jnp.float32),
        name="tc_final",
    )(sp, p, dis, dis2, b)


def kernel(x, edge_index, W0, b0, W1, b1, W2, b2):
    src = edge_index[0]
    dst = edge_index[1]
    z128 = jnp.zeros((RPT, D_HID), jnp.float32)
    o128 = jnp.ones((CH, D_HID), jnp.float32)

    deg_parts = _sc_degree(dst, z128, o128)
    dis, dis2, p0, g0 = _tc_stage0(deg_parts, x, W0)
    s0 = _edge128(g0, src, dst, z128)
    p1, g1 = _tc_combine(s0, p0, dis, dis2, b0.reshape(1, -1), W1)
    s1 = _edge128(g1, src, dst, z128)
    w2p = jnp.pad(W2, ((0, 0), (0, D_HID - D_OUT)))
    p2, g2 = _tc_combine(s1, p1, dis, dis2, b1.reshape(1, -1), w2p)
    s2 = _edge128(g2, src, dst, z128)
    return _tc_final(s2, p2, dis, dis2, b2.reshape(1, -1))


# speedup vs baseline: 1.2581x; 1.0960x over previous
import functools

import jax
import jax.numpy as jnp
from jax import lax
from jax.experimental import pallas as pl
from jax.experimental.pallas import tpu as pltpu
from jax.experimental.pallas import tpu_sc as plsc

N = 10000
E = 320000
D_IN = 128
D_HID = 128
D_OUT = 64

NC = 2
NS = 16
NW = NC * NS
EPT = E // NW
CH = 80
NCH = EPT // CH
NBUF = 3
NIB = 6
RPT = 632
RPT_LAST = N - (NS - 1) * RPT

_mesh = plsc.VectorSubcoreMesh(core_axis_name="c", subcore_axis_name="s")


def _zero_acc(z_hbm, acc, s):
    @pl.when(s < NS - 1)
    def _():
        pltpu.sync_copy(z_hbm, acc.at[pl.ds(s * RPT, RPT)])

    @pl.when(s == NS - 1)
    def _():
        pltpu.sync_copy(z_hbm.at[pl.ds(0, RPT_LAST)],
                        acc.at[pl.ds(s * RPT, RPT_LAST)])


def _acc_to_out(acc, out_hbm, c, s):
    @pl.when(s < NS - 1)
    def _():
        pltpu.sync_copy(acc.at[pl.ds(s * RPT, RPT)],
                        out_hbm.at[c].at[pl.ds(s * RPT, RPT)])

    @pl.when(s == NS - 1)
    def _():
        pltpu.sync_copy(acc.at[pl.ds(s * RPT, RPT_LAST)],
                        out_hbm.at[c].at[pl.ds(s * RPT, RPT_LAST)])


def _make_edge_kernel(d):
    scratch = [pltpu.VMEM_SHARED((N, d), jnp.float32)]
    scratch += [pltpu.VMEM((CH,), jnp.int32)] * NIB
    scratch += [pltpu.VMEM((CH,), jnp.int32)] * NIB
    scratch += [pltpu.VMEM((CH, d), jnp.float32)] * NBUF
    scratch += [pltpu.SemaphoreType.DMA] * (NIB + NBUF + NBUF)

    @functools.partial(
        pl.kernel,
        out_type=jax.ShapeDtypeStruct((NC, N, d), jnp.float32),
        mesh=_mesh,
        scratch_types=scratch,
        name=f"sc_edge_agg_{d}",
    )
    def edge_kernel(g_hbm, src_hbm, dst_hbm, z_hbm, out_hbm, acc, *bufs):
        sv = bufs[0:NIB]
        dv = bufs[NIB:2 * NIB]
        rv = bufs[2 * NIB:2 * NIB + NBUF]
        si = bufs[2 * NIB + NBUF:2 * NIB + NBUF + NIB]
        sg = bufs[2 * NIB + NBUF + NIB:2 * NIB + NBUF + NIB + NBUF]
        ss = bufs[2 * NIB + NBUF + NIB + NBUF:]
        c = lax.axis_index("c")
        s = lax.axis_index("s")
        wid = s * NC + c
        base = wid * EPT

        def idx_fetch(j, ib):
            pltpu.async_copy(src_hbm.at[pl.ds(base + j * CH, CH)],
                             sv[ib], si[ib])
            pltpu.async_copy(dst_hbm.at[pl.ds(base + j * CH, CH)],
                             dv[ib], si[ib])

        def idx_wait(j, ib):
            pltpu.make_async_copy(src_hbm.at[pl.ds(base + j * CH, CH)],
                                  sv[ib], si[ib]).wait()
            pltpu.make_async_copy(dst_hbm.at[pl.ds(base + j * CH, CH)],
                                  dv[ib], si[ib]).wait()

        _zero_acc(z_hbm, acc, s)
        for k in range(NBUF):
            idx_fetch(k, k)
        plsc.subcore_barrier()

        def step(j, k):
            b = k % NBUF
            ib = k % NIB
            pb = (k - 1) % NBUF
            pib = (k - 1) % NIB
            fib = (k + NBUF) % NIB

            @pl.when(j >= NBUF)
            def _():
                pltpu.make_async_copy(rv[b], acc.at[dv[(k - NBUF) % NIB]],
                                      ss[b]).wait()

            @pl.when(j + NBUF < NCH)
            def _():
                idx_fetch(j + NBUF, fib)

            idx_wait(j, ib)
            pltpu.async_copy(g_hbm.at[sv[ib]], rv[b], sg[b])

            @pl.when(j >= 1)
            def _():
                pltpu.make_async_copy(g_hbm.at[sv[pib]], rv[pb],
                                      sg[pb]).wait()
                pltpu.async_copy(rv[pb], acc.at[dv[pib]], ss[pb], add=True)

        def outer(j2, carry):
            for k in range(NIB):
                j = j2 * NIB + k

                @pl.when(j < NCH)
                def _():
                    step(j, k)
            return carry

        lax.fori_loop(0, (NCH + NIB - 1) // NIB, outer, 0)

        last = NCH - 1
        lb, lib = last % NBUF, last % NIB
        pltpu.make_async_copy(g_hbm.at[sv[lib]], rv[lb], sg[lb]).wait()
        pltpu.async_copy(rv[lb], acc.at[dv[lib]], ss[lb], add=True)
        for jj in range(NCH - NBUF, NCH):
            pltpu.make_async_copy(rv[jj % NBUF], acc.at[dv[jj % NIB]],
                                  ss[jj % NBUF]).wait()

        plsc.subcore_barrier()
        _acc_to_out(acc, out_hbm, c, s)

    return edge_kernel


_edge128 = _make_edge_kernel(D_HID)


_DEG_DEPTH = 4
_DEG_NIB = 8


@functools.partial(
    pl.kernel,
    out_type=jax.ShapeDtypeStruct((NC, N, D_HID), jnp.float32),
    mesh=_mesh,
    scratch_types=(
        [pltpu.VMEM_SHARED((N, D_HID), jnp.float32)]
        + [pltpu.VMEM((CH,), jnp.int32)] * _DEG_NIB
        + [pltpu.VMEM((CH, D_HID), jnp.float32)]
        + [pltpu.SemaphoreType.DMA] * (2 * _DEG_NIB)
    ),
    name="sc_degree",
)
def _sc_degree(dst_hbm, z_hbm, o_hbm, out_hbm, acc, *bufs):
    dv = bufs[0:_DEG_NIB]
    ones_v = bufs[_DEG_NIB]
    si = bufs[_DEG_NIB + 1:2 * _DEG_NIB + 1]
    ss = bufs[2 * _DEG_NIB + 1:]
    c = lax.axis_index("c")
    s = lax.axis_index("s")
    wid = s * NC + c
    base = wid * EPT

    def idx_fetch(j, ib):
        pltpu.async_copy(dst_hbm.at[pl.ds(base + j * CH, CH)], dv[ib], si[ib])

    pltpu.sync_copy(o_hbm, ones_v)
    _zero_acc(z_hbm, acc, s)
    for k in range(_DEG_DEPTH):
        idx_fetch(k, k)
    plsc.subcore_barrier()

    def step(j, k):
        ib = k % _DEG_NIB

        @pl.when(j >= _DEG_DEPTH)
        def _():
            pltpu.make_async_copy(
                ones_v, acc.at[dv[(k - _DEG_DEPTH) % _DEG_NIB]],
                ss[(k - _DEG_DEPTH) % _DEG_NIB]).wait()

        @pl.when(j + _DEG_DEPTH < NCH)
        def _():
            idx_fetch(j + _DEG_DEPTH, (k + _DEG_DEPTH) % _DEG_NIB)

        pltpu.make_async_copy(dst_hbm.at[pl.ds(base + j * CH, CH)],
                              dv[ib], si[ib]).wait()
        pltpu.async_copy(ones_v, acc.at[dv[ib]], ss[ib], add=True)

    def outer(j2, carry):
        for k in range(_DEG_NIB):
            j = j2 * _DEG_NIB + k

            @pl.when(j < NCH)
            def _():
                step(j, k)
        return carry

    lax.fori_loop(0, (NCH + _DEG_NIB - 1) // _DEG_NIB, outer, 0)
    for jj in range(NCH - _DEG_DEPTH, NCH):
        pltpu.make_async_copy(ones_v, acc.at[dv[jj % _DEG_NIB]],
                              ss[jj % _DEG_NIB]).wait()

    plsc.subcore_barrier()
    _acc_to_out(acc, out_hbm, c, s)


_BLK = 1000


def _matmul0_body(x_ref, w_ref, p_ref):
    p_ref[...] = jnp.dot(x_ref[...], w_ref[...],
                         preferred_element_type=jnp.float32)


def _tc_matmul0(x, w0):
    return pl.pallas_call(
        _matmul0_body,
        grid=(N // _BLK,),
        in_specs=[
            pl.BlockSpec((_BLK, D_IN), lambda i: (i, 0)),
            pl.BlockSpec((D_IN, D_HID), lambda i: (0, 0)),
        ],
        out_specs=pl.BlockSpec((_BLK, D_HID), lambda i: (i, 0)),
        out_shape=jax.ShapeDtypeStruct((N, D_HID), jnp.float32),
        name="tc_matmul0",
    )(x, w0)


def _stage0_body(degp_ref, p_ref, dis_ref, dis2_ref, g_ref):
    deg = degp_ref[0, :, :1] + degp_ref[1, :, :1] + 1.0
    dis = lax.rsqrt(deg)
    dis_ref[...] = dis
    dis2_ref[...] = dis * dis
    g_ref[...] = p_ref[...] * dis


def _tc_stage0(deg_parts, p):
    return pl.pallas_call(
        _stage0_body,
        grid=(N // _BLK,),
        in_specs=[
            pl.BlockSpec((NC, _BLK, D_HID), lambda i: (0, i, 0)),
            pl.BlockSpec((_BLK, D_HID), lambda i: (i, 0)),
        ],
        out_specs=[
            pl.BlockSpec((_BLK, 1), lambda i: (i, 0)),
            pl.BlockSpec((_BLK, 1), lambda i: (i, 0)),
            pl.BlockSpec((_BLK, D_HID), lambda i: (i, 0)),
        ],
        out_shape=[
            jax.ShapeDtypeStruct((N, 1), jnp.float32),
            jax.ShapeDtypeStruct((N, 1), jnp.float32),
            jax.ShapeDtypeStruct((N, D_HID), jnp.float32),
        ],
        name="tc_stage0",
    )(deg_parts, p)


def _combine_body(sp_ref, p_ref, dis_ref, dis2_ref, b_ref, w_ref,
                  pn_ref, gn_ref):
    S = sp_ref[0] + sp_ref[1]
    dis = dis_ref[...]
    h = S * dis + p_ref[...] * dis2_ref[...] + b_ref[...]
    h = jnp.maximum(h, 0.0)
    pn = jnp.dot(h, w_ref[...], preferred_element_type=jnp.float32)
    pn_ref[...] = pn
    gn_ref[...] = pn * dis


def _tc_combine(sp, p, dis, dis2, b, w):
    d_in = p.shape[1]
    d_out = w.shape[1]
    return pl.pallas_call(
        _combine_body,
        grid=(N // _BLK,),
        in_specs=[
            pl.BlockSpec((NC, _BLK, d_in), lambda i: (0, i, 0)),
            pl.BlockSpec((_BLK, d_in), lambda i: (i, 0)),
            pl.BlockSpec((_BLK, 1), lambda i: (i, 0)),
            pl.BlockSpec((_BLK, 1), lambda i: (i, 0)),
            pl.BlockSpec((1, d_in), lambda i: (0, 0)),
            pl.BlockSpec((d_in, d_out), lambda i: (0, 0)),
        ],
        out_specs=[
            pl.BlockSpec((_BLK, d_out), lambda i: (i, 0)),
            pl.BlockSpec((_BLK, d_out), lambda i: (i, 0)),
        ],
        out_shape=[
            jax.ShapeDtypeStruct((N, d_out), jnp.float32),
            jax.ShapeDtypeStruct((N, d_out), jnp.float32),
        ],
        name="tc_combine",
    )(sp, p, dis, dis2, b, w)


def _final_body(sp_ref, p_ref, dis_ref, dis2_ref, b_ref, o_ref):
    S = sp_ref[0] + sp_ref[1]
    full = S * dis_ref[...] + p_ref[...] * dis2_ref[...]
    o_ref[...] = full[:, :D_OUT] + b_ref[...]


def _tc_final(sp, p, dis, dis2, b):
    return pl.pallas_call(
        _final_body,
        grid=(N // _BLK,),
        in_specs=[
            pl.BlockSpec((NC, _BLK, D_HID), lambda i: (0, i, 0)),
            pl.BlockSpec((_BLK, D_HID), lambda i: (i, 0)),
            pl.BlockSpec((_BLK, 1), lambda i: (i, 0)),
            pl.BlockSpec((_BLK, 1), lambda i: (i, 0)),
            pl.BlockSpec((1, D_OUT), lambda i: (0, 0)),
        ],
        out_specs=pl.BlockSpec((_BLK, D_OUT), lambda i: (i, 0)),
        out_shape=jax.ShapeDtypeStruct((N, D_OUT), jnp.float32),
        name="tc_final",
    )(sp, p, dis, dis2, b)


def kernel(x, edge_index, W0, b0, W1, b1, W2, b2):
    src = edge_index[0]
    dst = edge_index[1]
    z128 = jnp.zeros((RPT, D_HID), jnp.float32)
    o128 = jnp.ones((CH, D_HID), jnp.float32)

    p0 = _tc_matmul0(x, W0)
    deg_parts = _sc_degree(dst, z128, o128)
    dis, dis2, g0 = _tc_stage0(deg_parts, p0)
    s0 = _edge128(g0, src, dst, z128)
    p1, g1 = _tc_combine(s0, p0, dis, dis2, b0.reshape(1, -1), W1)
    s1 = _edge128(g1, src, dst, z128)
    w2p = jnp.pad(W2, ((0, 0), (0, D_HID - D_OUT)))
    p2, g2 = _tc_combine(s1, p1, dis, dis2, b1.reshape(1, -1), w2p)
    s2 = _edge128(g2, src, dst, z128)
    return _tc_final(s2, p2, dis, dis2, b2.reshape(1, -1))
